# Initial kernel scaffold; baseline (speedup 1.0000x reference)
import jax
import jax.numpy as jnp
from jax.experimental import pallas as pl


def kernel(inputs, embeddings):
    raise NotImplementedError("write your pallas kernel here")


# speedup vs baseline: 1.4575x; 1.4575x over previous
import functools

import jax
import jax.numpy as jnp
from jax import lax
from jax.experimental import pallas as pl
from jax.experimental.pallas import tpu as pltpu
from jax.experimental.pallas import tpu_sc as plsc

_INPUT_DIM = 1000000
_OUT_DIM = 32
_BATCH = 4096
_SEQ = 200

_B = _BATCH * _SEQ
_NC = 2
_NS = 16
_NW = _NC * _NS
_BPW = _B // _NW
_C = 1024
_NCHUNK = _BPW // _C


def _make_gather():
    mesh = plsc.VectorSubcoreMesh(core_axis_name="c", subcore_axis_name="s")

    @functools.partial(
        pl.kernel,
        out_type=jax.ShapeDtypeStruct((_B, _OUT_DIM), jnp.float32),
        mesh=mesh,
        scratch_types=[
            pltpu.VMEM((_C,), jnp.int32),
            pltpu.VMEM((_C, _OUT_DIM), jnp.float32),
            pltpu.SemaphoreType.DMA,
        ],
        compiler_params=pltpu.CompilerParams(use_tc_tiling_on_sc=False),
    )
    def gather_kernel(idx_hbm, table_hbm, out_hbm, idx_v, rows_v, sem):
        wid = lax.axis_index("s") * _NC + lax.axis_index("c")
        base = wid * _BPW

        def body(j, carry):
            off = base + j * _C
            pltpu.sync_copy(idx_hbm.at[pl.ds(off, _C)], idx_v)
            pltpu.async_copy(table_hbm.at[idx_v], rows_v, sem).wait()
            pltpu.sync_copy(rows_v, out_hbm.at[pl.ds(off, _C)])
            return carry

        lax.fori_loop(0, _NCHUNK, body, 0)

    return gather_kernel


_gather = _make_gather()


@jax.jit
def kernel(inputs, embeddings):
    flat = inputs.reshape(-1).astype(jnp.int32)
    out = _gather(flat, embeddings)
    return out.reshape(_BATCH, _SEQ, _OUT_DIM)


# speedup vs baseline: 1.4995x; 1.0288x over previous
import functools

import jax
import jax.numpy as jnp
from jax import lax
from jax.experimental import pallas as pl
from jax.experimental.pallas import tpu as pltpu
from jax.experimental.pallas import tpu_sc as plsc

_INPUT_DIM = 1000000
_OUT_DIM = 32
_BATCH = 4096
_SEQ = 200

_B = _BATCH * _SEQ
_NC = 2
_NS = 16
_NW = _NC * _NS
_BPW = _B // _NW
_C = 1280
_NCHUNK = _BPW // _C


def _make_gather():
    mesh = plsc.VectorSubcoreMesh(core_axis_name="c", subcore_axis_name="s")

    @functools.partial(
        pl.kernel,
        out_type=jax.ShapeDtypeStruct((_B, _OUT_DIM), jnp.float32),
        mesh=mesh,
        scratch_types=[
            pltpu.VMEM((_BPW,), jnp.int32),
            pltpu.VMEM((2, _C, _OUT_DIM), jnp.float32),
            pltpu.SemaphoreType.DMA,
            pltpu.SemaphoreType.DMA,
            pltpu.SemaphoreType.DMA,
            pltpu.SemaphoreType.DMA,
        ],
        compiler_params=pltpu.CompilerParams(use_tc_tiling_on_sc=False),
    )
    def gather_kernel(idx_hbm, table_hbm, out_hbm, idx_v, rows_v,
                      sg0, sg1, ss0, ss1):
        wid = lax.axis_index("s") * _NC + lax.axis_index("c")
        base = wid * _BPW

        pltpu.sync_copy(idx_hbm.at[pl.ds(base, _BPW)], idx_v)

        sg = (sg0, sg1)
        ss = (ss0, ss1)
        gathers = [None] * _NCHUNK
        stores = [None] * _NCHUNK

        def start_gather(j):
            b = j % 2
            gathers[j] = pltpu.async_copy(
                table_hbm.at[idx_v.at[pl.ds(j * _C, _C)]],
                rows_v.at[b], sg[b])

        def start_store(j):
            b = j % 2
            stores[j] = pltpu.async_copy(
                rows_v.at[b], out_hbm.at[pl.ds(base + j * _C, _C)], ss[b])

        start_gather(0)
        for j in range(1, _NCHUNK + 1):
            if j < _NCHUNK:
                if j >= 2:
                    stores[j - 2].wait()
                start_gather(j)
            gathers[j - 1].wait()
            start_store(j - 1)
        stores[_NCHUNK - 2].wait()
        stores[_NCHUNK - 1].wait()

    return gather_kernel


_gather = _make_gather()


@jax.jit
def kernel(inputs, embeddings):
    flat = inputs.reshape(-1).astype(jnp.int32)
    out = _gather(flat, embeddings)
    return out.reshape(_BATCH, _SEQ, _OUT_DIM)
